# baseline (speedup 1.0000x reference)
import jax
import jax.numpy as jnp
from jax import lax
from jax.experimental import pallas as pl
from jax.experimental.pallas import tpu as pltpu
from jax.experimental.pallas import tpu_sc as plsc

NC = 2
NS = 16
NW = NC * NS

IDX_ROW = 128
ROWS_PER_CHUNK = 2
CHUNK = ROWS_PER_CHUNK * IDX_ROW
NBUF = 2


def _gather_body(idx_hbm, tab_hbm, out_hbm,
                 idx_all, out_v0, out_v1,
                 gsem0, gsem1, wsem0, wsem1):
    wid = lax.axis_index("s") * NC + lax.axis_index("c")
    total_rows = idx_hbm.shape[0]
    rows_per_w = total_rows // NW
    n_chunks = rows_per_w // ROWS_PER_CHUNK
    n_outer = n_chunks // NBUF
    row_base0 = wid * rows_per_w

    out_bufs = (out_v0, out_v1)
    gsems = (gsem0, gsem1)
    wsems = (wsem0, wsem1)

    pltpu.sync_copy(idx_hbm.at[pl.ds(row_base0, rows_per_w)], idx_all)

    def outer(k, carry):
        gdescs = [[] for _ in range(NBUF)]
        for b in range(NBUF):
            local_row = (k * NBUF + b) * ROWS_PER_CHUNK

            @pl.when(k > 0)
            def _drain(b=b):
                pltpu.make_async_copy(
                    out_hbm.at[pl.ds(0, CHUNK)], out_bufs[b], wsems[b]).wait()

            for j in range(ROWS_PER_CHUNK):
                gdescs[b].append(pltpu.async_copy(
                    tab_hbm.at[idx_all.at[local_row + j]],
                    out_bufs[b].at[pl.ds(j * IDX_ROW, IDX_ROW)], gsems[b]))
        for b in range(NBUF):
            local_row = (k * NBUF + b) * ROWS_PER_CHUNK
            for d in gdescs[b]:
                d.wait()
            pltpu.async_copy(
                out_bufs[b],
                out_hbm.at[pl.ds((row_base0 + local_row) * IDX_ROW, CHUNK)],
                wsems[b])
        return carry

    lax.fori_loop(0, n_outer, outer, 0)
    for b in range(NBUF):
        pltpu.make_async_copy(
            out_hbm.at[pl.ds(0, CHUNK)], out_bufs[b], wsems[b]).wait()


def kernel(log_seqs, time1_seqs, time2_seqs, item_pop1, item_pop2):
    batch, hist = log_seqs.shape
    d1 = item_pop1.shape[1]
    d2 = item_pop2.shape[1]
    d = d1 + d2
    n_idx = batch * hist
    idx2d = log_seqs.astype(jnp.int32).reshape(n_idx // IDX_ROW, IDX_ROW)
    tabcat = jnp.concatenate([item_pop1, item_pop2], axis=-1)

    mesh = plsc.VectorSubcoreMesh(core_axis_name="c", subcore_axis_name="s",
                                  num_cores=NC, num_subcores=NS)
    run = pl.kernel(
        _gather_body,
        out_type=jax.ShapeDtypeStruct((n_idx, d), jnp.float32),
        mesh=mesh,
        scratch_types=[
            pltpu.VMEM((n_idx // IDX_ROW // NW, IDX_ROW), jnp.int32),
            pltpu.VMEM((CHUNK, d), jnp.float32),
            pltpu.VMEM((CHUNK, d), jnp.float32),
            pltpu.SemaphoreType.DMA,
            pltpu.SemaphoreType.DMA,
            pltpu.SemaphoreType.DMA,
            pltpu.SemaphoreType.DMA,
        ],
    )
    out = run(idx2d, tabcat)
    return out.reshape(batch, hist, d)


# speedup vs baseline: 1.0061x; 1.0061x over previous
import jax
import jax.numpy as jnp
from jax import lax
from jax.experimental import pallas as pl
from jax.experimental.pallas import tpu as pltpu
from jax.experimental.pallas import tpu_sc as plsc

NC = 2
NS = 16
NW = NC * NS

IDX_ROW = 128
ROWS_PER_CHUNK = 2
CHUNK = ROWS_PER_CHUNK * IDX_ROW
NBUF = 3


def _gather_body(idx_hbm, tab_hbm, out_hbm,
                 idx_all, out_v0, out_v1, out_v2,
                 gsem0, gsem1, gsem2, wsem0, wsem1, wsem2):
    wid = lax.axis_index("s") * NC + lax.axis_index("c")
    total_rows = idx_hbm.shape[0]
    rows_per_w = total_rows // NW
    n_chunks = rows_per_w // ROWS_PER_CHUNK
    n_outer = n_chunks // NBUF
    row_base0 = wid * rows_per_w

    out_bufs = (out_v0, out_v1, out_v2)
    gsems = (gsem0, gsem1, gsem2)
    wsems = (wsem0, wsem1, wsem2)
    n_tail = n_chunks - n_outer * NBUF

    pltpu.sync_copy(idx_hbm.at[pl.ds(row_base0, rows_per_w)], idx_all)

    def outer(k, carry):
        gdescs = [[] for _ in range(NBUF)]
        for b in range(NBUF):
            local_row = (k * NBUF + b) * ROWS_PER_CHUNK

            @pl.when(k > 0)
            def _drain(b=b):
                pltpu.make_async_copy(
                    out_hbm.at[pl.ds(0, CHUNK)], out_bufs[b], wsems[b]).wait()

            for j in range(ROWS_PER_CHUNK):
                gdescs[b].append(pltpu.async_copy(
                    tab_hbm.at[idx_all.at[local_row + j]],
                    out_bufs[b].at[pl.ds(j * IDX_ROW, IDX_ROW)], gsems[b]))
        for b in range(NBUF):
            local_row = (k * NBUF + b) * ROWS_PER_CHUNK
            for d in gdescs[b]:
                d.wait()
            pltpu.async_copy(
                out_bufs[b],
                out_hbm.at[pl.ds((row_base0 + local_row) * IDX_ROW, CHUNK)],
                wsems[b])
        return carry

    lax.fori_loop(0, n_outer, outer, 0)

    for b in range(n_tail):
        local_row = (n_outer * NBUF + b) * ROWS_PER_CHUNK
        pltpu.make_async_copy(
            out_hbm.at[pl.ds(0, CHUNK)], out_bufs[b], wsems[b]).wait()
        descs = [pltpu.async_copy(
            tab_hbm.at[idx_all.at[local_row + j]],
            out_bufs[b].at[pl.ds(j * IDX_ROW, IDX_ROW)], gsems[b])
            for j in range(ROWS_PER_CHUNK)]
        for dsc in descs:
            dsc.wait()
        pltpu.async_copy(
            out_bufs[b],
            out_hbm.at[pl.ds((row_base0 + local_row) * IDX_ROW, CHUNK)],
            wsems[b])

    for b in range(NBUF):
        pltpu.make_async_copy(
            out_hbm.at[pl.ds(0, CHUNK)], out_bufs[b], wsems[b]).wait()


def kernel(log_seqs, time1_seqs, time2_seqs, item_pop1, item_pop2):
    batch, hist = log_seqs.shape
    d1 = item_pop1.shape[1]
    d2 = item_pop2.shape[1]
    d = d1 + d2
    n_idx = batch * hist
    idx2d = log_seqs.astype(jnp.int32).reshape(n_idx // IDX_ROW, IDX_ROW)
    tabcat = jnp.concatenate([item_pop1, item_pop2], axis=-1)

    mesh = plsc.VectorSubcoreMesh(core_axis_name="c", subcore_axis_name="s",
                                  num_cores=NC, num_subcores=NS)
    run = pl.kernel(
        _gather_body,
        out_type=jax.ShapeDtypeStruct((n_idx, d), jnp.float32),
        mesh=mesh,
        scratch_types=[
            pltpu.VMEM((n_idx // IDX_ROW // NW, IDX_ROW), jnp.int32),
            pltpu.VMEM((CHUNK, d), jnp.float32),
            pltpu.VMEM((CHUNK, d), jnp.float32),
            pltpu.VMEM((CHUNK, d), jnp.float32),
            pltpu.SemaphoreType.DMA,
            pltpu.SemaphoreType.DMA,
            pltpu.SemaphoreType.DMA,
            pltpu.SemaphoreType.DMA,
            pltpu.SemaphoreType.DMA,
            pltpu.SemaphoreType.DMA,
        ],
    )
    out = run(idx2d, tabcat)
    return out.reshape(batch, hist, d)


# speedup vs baseline: 1.0087x; 1.0026x over previous
import jax
import jax.numpy as jnp
from jax import lax
from jax.experimental import pallas as pl
from jax.experimental.pallas import tpu as pltpu
from jax.experimental.pallas import tpu_sc as plsc

NC = 2
NS = 16
NW = NC * NS

IDX_ROW = 128
ROWS_PER_CHUNK = 1
CHUNK = ROWS_PER_CHUNK * IDX_ROW
NBUF = 4


def _gather_body(idx_hbm, tab_hbm, out_hbm, idx_all, *bufs_and_sems):
    out_bufs = bufs_and_sems[:NBUF]
    gsems = bufs_and_sems[NBUF:2 * NBUF]
    wsems = bufs_and_sems[2 * NBUF:]

    wid = lax.axis_index("s") * NC + lax.axis_index("c")
    total_rows = idx_hbm.shape[0]
    rows_per_w = total_rows // NW
    n_chunks = rows_per_w // ROWS_PER_CHUNK
    n_outer = n_chunks // NBUF
    n_tail = n_chunks - n_outer * NBUF
    row_base0 = wid * rows_per_w

    pltpu.sync_copy(idx_hbm.at[pl.ds(row_base0, rows_per_w)], idx_all)

    def fire(b, local_row):
        return [pltpu.async_copy(
            tab_hbm.at[idx_all.at[local_row + j]],
            out_bufs[b].at[pl.ds(j * IDX_ROW, IDX_ROW)], gsems[b])
            for j in range(ROWS_PER_CHUNK)]

    def write(b, local_row):
        pltpu.async_copy(
            out_bufs[b],
            out_hbm.at[pl.ds((row_base0 + local_row) * IDX_ROW, CHUNK)],
            wsems[b])

    def drain_write(b):
        pltpu.make_async_copy(
            out_hbm.at[pl.ds(0, CHUNK)], out_bufs[b], wsems[b]).wait()

    def outer(k, carry):
        gdescs = []
        for b in range(NBUF):
            local_row = (k * NBUF + b) * ROWS_PER_CHUNK

            @pl.when(k > 0)
            def _drain(b=b):
                drain_write(b)

            gdescs.append(fire(b, local_row))
        for b in range(NBUF):
            local_row = (k * NBUF + b) * ROWS_PER_CHUNK
            for dsc in gdescs[b]:
                dsc.wait()
            write(b, local_row)
        return carry

    lax.fori_loop(0, n_outer, outer, 0)

    for b in range(n_tail):
        local_row = (n_outer * NBUF + b) * ROWS_PER_CHUNK
        drain_write(b)
        for dsc in fire(b, local_row):
            dsc.wait()
        write(b, local_row)

    for b in range(NBUF):
        drain_write(b)


def kernel(log_seqs, time1_seqs, time2_seqs, item_pop1, item_pop2):
    batch, hist = log_seqs.shape
    d1 = item_pop1.shape[1]
    d2 = item_pop2.shape[1]
    d = d1 + d2
    n_idx = batch * hist
    idx2d = log_seqs.astype(jnp.int32).reshape(n_idx // IDX_ROW, IDX_ROW)
    tabcat = jnp.concatenate([item_pop1, item_pop2], axis=-1)

    mesh = plsc.VectorSubcoreMesh(core_axis_name="c", subcore_axis_name="s",
                                  num_cores=NC, num_subcores=NS)
    run = pl.kernel(
        _gather_body,
        out_type=jax.ShapeDtypeStruct((n_idx, d), jnp.float32),
        mesh=mesh,
        scratch_types=(
            [pltpu.VMEM((n_idx // IDX_ROW // NW, IDX_ROW), jnp.int32)]
            + [pltpu.VMEM((CHUNK, d), jnp.float32)] * NBUF
            + [pltpu.SemaphoreType.DMA] * (2 * NBUF)
        ),
    )
    out = run(idx2d, tabcat)
    return out.reshape(batch, hist, d)
